# baseline (speedup 1.0000x reference)
import functools

import jax
import jax.numpy as jnp
from jax import lax
from jax.experimental import pallas as pl
from jax.experimental.pallas import tpu as pltpu
from jax.experimental.pallas import tpu_sc as plsc

_NBUF = 4


def _emb_kernel(bsz, seq, d, n_workers, num_cores, num_subcores, chunk):
    n_per_w = (bsz * seq) // n_workers
    w_per_b = n_workers // bsz
    n_chunks = n_per_w // chunk
    assert n_chunks % _NBUF == 0 and n_chunks >= 2 * _NBUF

    mesh = plsc.VectorSubcoreMesh(core_axis_name="c", subcore_axis_name="s")

    @functools.partial(
        pl.kernel,
        mesh=mesh,
        out_type=jax.ShapeDtypeStruct((bsz, seq, d), jnp.float32),
        scratch_types=[
            pltpu.VMEM((n_per_w,), jnp.int32),
            pltpu.VMEM((_NBUF, chunk, d), jnp.float32),
            pltpu.VMEM_SHARED((num_subcores, 2, chunk, d), jnp.float32),
        ]
        + [pltpu.SemaphoreType.DMA] * (_NBUF + 4),
    )
    def emb(idx_hbm, table_hbm, out_hbm, idx_v, rows_v, sp_v, *sems):
        sg = sems[:_NBUF]
        sc_ = sems[_NBUF : _NBUF + 2]
        sp = sems[_NBUF + 2 :]
        cid = lax.axis_index("c")
        sid = lax.axis_index("s")
        wid = sid * num_cores + cid
        batch = wid // w_per_b
        off = (wid % w_per_b) * n_per_w
        pltpu.sync_copy(idx_hbm.at[batch, pl.ds(off, n_per_w)], idx_v)

        def gather(i, b):
            return pltpu.make_async_copy(
                table_hbm.at[idx_v.at[pl.ds(i * chunk, chunk)]],
                rows_v.at[b],
                sg[b],
            )

        def xbar(b, m):
            return pltpu.make_async_copy(rows_v.at[b], sp_v.at[sid, m], sc_[m])

        def put(i, m):
            return pltpu.make_async_copy(
                sp_v.at[sid, m],
                out_hbm.at[batch, pl.ds(off + i * chunk, chunk)],
                sp[m],
            )

        gather(0, 0).start()
        gather(1, 1).start()
        gather(2, 2).start()

        def body(k, carry):
            for b in range(_NBUF):
                i = _NBUF * k + b
                pb = (b + 3) % _NBUF
                m = b % 2
                pm = 1 - m

                @pl.when(i >= 2)
                def _():
                    put(i - 2, m).wait()

                gather(i, b).wait()
                xbar(b, m).start()

                @pl.when(i >= 1)
                def _():
                    xbar(pb, pm).wait()
                    put(i - 1, pm).start()

                @pl.when(i + 3 < n_chunks)
                def _():
                    gather(i + 3, pb).start()
            return carry

        lax.fori_loop(0, n_chunks // _NBUF, body, 0)
        last_m = (n_chunks - 1) % 2
        xbar((n_chunks - 1) % _NBUF, last_m).wait()
        put(n_chunks - 1, last_m).start()
        put(n_chunks - 2, 1 - last_m).wait()
        put(n_chunks - 1, last_m).wait()

    return emb


def kernel(input_ids, table):
    b, s = input_ids.shape
    v, d = table.shape
    idx = input_ids.astype(jnp.int32)
    info = plsc.get_sparse_core_info()
    nw = info.num_cores * info.num_subcores
    emb = _emb_kernel(
        b, s, d, nw, info.num_cores, info.num_subcores, chunk=8
    )
    return emb(idx, table)


# speedup vs baseline: 1.0128x; 1.0128x over previous
import functools

import jax
import jax.numpy as jnp
from jax import lax
from jax.experimental import pallas as pl
from jax.experimental.pallas import tpu as pltpu
from jax.experimental.pallas import tpu_sc as plsc

_NBUF = 4


def _emb_kernel(bsz, seq, d, n_workers, num_cores, chunk):
    n_per_w = (bsz * seq) // n_workers
    w_per_b = n_workers // bsz
    n_chunks = n_per_w // chunk
    assert n_chunks % _NBUF == 0 and n_chunks >= 2 * _NBUF

    mesh = plsc.VectorSubcoreMesh(core_axis_name="c", subcore_axis_name="s")

    @functools.partial(
        pl.kernel,
        mesh=mesh,
        out_type=jax.ShapeDtypeStruct((bsz, seq, d), jnp.float32),
        scratch_types=[
            pltpu.VMEM((n_per_w,), jnp.int32),
            pltpu.VMEM((_NBUF, chunk, d), jnp.float32),
        ]
        + [pltpu.SemaphoreType.DMA] * (2 * _NBUF),
    )
    def emb(idx_hbm, table_hbm, out_hbm, idx_v, rows_v, *sems):
        sin = sems[:_NBUF]
        sout = sems[_NBUF:]
        wid = lax.axis_index("s") * num_cores + lax.axis_index("c")
        batch = wid // w_per_b
        off = (wid % w_per_b) * n_per_w
        pltpu.sync_copy(idx_hbm.at[batch, pl.ds(off, n_per_w)], idx_v)

        def gather(i, b):
            return pltpu.make_async_copy(
                table_hbm.at[idx_v.at[pl.ds(i * chunk, chunk)]],
                rows_v.at[b],
                sin[b],
            )

        def put(i, b):
            return pltpu.make_async_copy(
                rows_v.at[b],
                out_hbm.at[batch, pl.ds(off + i * chunk, chunk)],
                sout[b],
            )

        gather(0, 0).start()
        gather(1, 1).start()

        def body(k, carry):
            for b in range(_NBUF):
                i = _NBUF * k + b
                nb = (b + 2) % _NBUF

                @pl.when(i >= 2)
                def _():
                    put(i - 2, nb).wait()

                @pl.when(i + 2 < n_chunks)
                def _():
                    gather(i + 2, nb).start()

                gather(i, b).wait()
                put(i, b).start()
            return carry

        lax.fori_loop(0, n_chunks // _NBUF, body, 0)
        put(n_chunks - 2, (n_chunks - 2) % _NBUF).wait()
        put(n_chunks - 1, (n_chunks - 1) % _NBUF).wait()

    return emb


def kernel(input_ids, table):
    b, s = input_ids.shape
    v, d = table.shape
    idx = input_ids.astype(jnp.int32)
    info = plsc.get_sparse_core_info()
    nw = info.num_cores * info.num_subcores
    emb = _emb_kernel(b, s, d, nw, info.num_cores, chunk=8)
    return emb(idx, table)
